# Initial kernel scaffold; baseline (speedup 1.0000x reference)
import jax
import jax.numpy as jnp
from jax.experimental import pallas as pl


def kernel(hidden_states, W1, b1, W2, b2):
    raise NotImplementedError("write your pallas kernel here")


# speedup vs baseline: 1.0811x; 1.0811x over previous
import jax
import jax.numpy as jnp
from jax.experimental import pallas as pl
from jax.experimental.pallas import tpu as pltpu

_M_T = 2048
_N_T = 1024
_K_T = 512


def _gate_kernel(x_ref, w1_ref, b1_ref, w2_ref, b2_ref,
                 weights_out, logits_out, ent_out, cv_out,
                 h_acc, l_acc, ent_acc, sw_acc, sw2_acc,
                 *, nm, nn, nk, tokens, gates):
    m = pl.program_id(0)
    n = pl.program_id(1)
    k = pl.program_id(2)

    xb = x_ref[...].astype(jnp.bfloat16)
    wb = w1_ref[...].astype(jnp.bfloat16)
    part = jnp.dot(xb, wb, preferred_element_type=jnp.float32)

    @pl.when(k == 0)
    def _():
        h_acc[...] = part

    @pl.when(k != 0)
    def _():
        h_acc[...] += part

    @pl.when(k == nk - 1)
    def _():
        h = h_acc[...] + b1_ref[...]
        g = jax.nn.gelu(h).astype(jnp.bfloat16)
        lpart = jnp.dot(g, w2_ref[...].astype(jnp.bfloat16),
                        preferred_element_type=jnp.float32)

        @pl.when(n == 0)
        def _():
            l_acc[...] = lpart

        @pl.when(n != 0)
        def _():
            l_acc[...] += lpart

        @pl.when(n == nn - 1)
        def _():
            logits = l_acc[...] + b2_ref[...]
            logits_out[...] = logits
            w = jax.nn.sigmoid(logits)
            weights_out[...] = w
            sw = jnp.sum(w, axis=0, keepdims=True)
            sw2 = jnp.sum(w * w, axis=0, keepdims=True)
            mx = jnp.max(logits, axis=-1, keepdims=True)
            e = jnp.exp(logits - mx)
            p = e / jnp.sum(e, axis=-1, keepdims=True)
            ent = -jnp.sum(p * jnp.log(p + 1e-9), axis=-1, keepdims=True)
            d = ent - jnp.log(jnp.float32(gates))
            e2 = jnp.sum(d * d).reshape(1, 1)

            @pl.when(m == 0)
            def _():
                sw_acc[...] = sw
                sw2_acc[...] = sw2
                ent_acc[...] = e2

            @pl.when(m != 0)
            def _():
                sw_acc[...] += sw
                sw2_acc[...] += sw2
                ent_acc[...] += e2

            @pl.when(m == nm - 1)
            def _():
                ent_out[...] = ent_acc[...] / tokens
                mean = sw_acc[...] / tokens
                var = sw2_acc[...] / tokens - mean * mean
                std = jnp.sqrt(jnp.maximum(var, 0.0))
                cv_out[...] = jnp.mean(std / (mean + 1e-9)).reshape(1, 1)


def kernel(hidden_states, W1, b1, W2, b2):
    B, S, H = hidden_states.shape
    G = W2.shape[1]
    M = B * S
    x = hidden_states.reshape(M, H)

    mt = min(_M_T, M)
    nt = min(_N_T, H)
    kt = min(_K_T, H)
    nm, nn, nk = M // mt, H // nt, H // kt

    b1r = b1.reshape(1, H)
    b2r = b2.reshape(1, G)

    out_shape = (
        jax.ShapeDtypeStruct((M, G), jnp.float32),
        jax.ShapeDtypeStruct((M, G), jnp.float32),
        jax.ShapeDtypeStruct((1, 1), jnp.float32),
        jax.ShapeDtypeStruct((1, 1), jnp.float32),
    )
    grid = (nm, nn, nk)

    import functools
    body = functools.partial(_gate_kernel, nm=nm, nn=nn, nk=nk,
                             tokens=float(M), gates=G)

    weights, logits, ent, cv = pl.pallas_call(
        body,
        grid=grid,
        in_specs=[
            pl.BlockSpec((mt, kt), lambda m, n, k: (m, k)),
            pl.BlockSpec((kt, nt), lambda m, n, k: (k, n)),
            pl.BlockSpec((1, nt), lambda m, n, k: (0, n)),
            pl.BlockSpec((nt, G), lambda m, n, k: (n, 0)),
            pl.BlockSpec((1, G), lambda m, n, k: (0, 0)),
        ],
        out_specs=[
            pl.BlockSpec((mt, G), lambda m, n, k: (m, 0)),
            pl.BlockSpec((mt, G), lambda m, n, k: (m, 0)),
            pl.BlockSpec((1, 1), lambda m, n, k: (0, 0)),
            pl.BlockSpec((1, 1), lambda m, n, k: (0, 0)),
        ],
        out_shape=out_shape,
        scratch_shapes=[
            pltpu.VMEM((mt, nt), jnp.float32),
            pltpu.VMEM((mt, G), jnp.float32),
            pltpu.VMEM((1, 1), jnp.float32),
            pltpu.VMEM((1, G), jnp.float32),
            pltpu.VMEM((1, G), jnp.float32),
        ],
        compiler_params=pltpu.CompilerParams(
            dimension_semantics=("arbitrary", "arbitrary", "arbitrary"),
        ),
    )(x, W1, b1r, W2, b2r)

    return (weights.reshape(B, S, G), logits.reshape(B, S, G),
            ent.reshape(()), cv.reshape(()))


# speedup vs baseline: 1.3318x; 1.2319x over previous
import jax
import jax.numpy as jnp
from jax.experimental import pallas as pl
from jax.experimental.pallas import tpu as pltpu

_M_T = 2048
_N_T = 1024
_K_T = 1024


def _gate_kernel(x_ref, w1_ref, b1_ref, w2_ref, b2_ref,
                 weights_out, logits_out, ent_out, cv_out,
                 h_acc, l_acc, ent_acc, sw_acc, sw2_acc,
                 *, nm, nn, nk, tokens, gates):
    m = pl.program_id(0)
    n = pl.program_id(1)
    k = pl.program_id(2)

    xb = x_ref[...].astype(jnp.bfloat16)
    wb = w1_ref[...].astype(jnp.bfloat16)
    part = jnp.dot(xb, wb, preferred_element_type=jnp.float32)

    @pl.when(k == 0)
    def _():
        h_acc[...] = part

    @pl.when(k != 0)
    def _():
        h_acc[...] += part

    @pl.when(k == nk - 1)
    def _():
        h = (h_acc[...] + b1_ref[...]).astype(jnp.bfloat16)
        g = jax.nn.gelu(h)
        lpart = jnp.dot(g, w2_ref[...].astype(jnp.bfloat16),
                        preferred_element_type=jnp.float32)

        @pl.when(n == 0)
        def _():
            l_acc[...] = lpart

        @pl.when(n != 0)
        def _():
            l_acc[...] += lpart

        @pl.when(n == nn - 1)
        def _():
            logits = l_acc[...] + b2_ref[...]
            logits_out[...] = logits
            w = jax.nn.sigmoid(logits)
            weights_out[...] = w
            sw = jnp.sum(w, axis=0, keepdims=True)
            sw2 = jnp.sum(w * w, axis=0, keepdims=True)
            mx = jnp.max(logits, axis=-1, keepdims=True)
            e = jnp.exp(logits - mx)
            p = e / jnp.sum(e, axis=-1, keepdims=True)
            ent = -jnp.sum(p * jnp.log(p + 1e-9), axis=-1, keepdims=True)
            d = ent - jnp.log(jnp.float32(gates))
            e2 = jnp.sum(d * d).reshape(1, 1)

            @pl.when(m == 0)
            def _():
                sw_acc[...] = sw
                sw2_acc[...] = sw2
                ent_acc[...] = e2

            @pl.when(m != 0)
            def _():
                sw_acc[...] += sw
                sw2_acc[...] += sw2
                ent_acc[...] += e2

            @pl.when(m == nm - 1)
            def _():
                ent_out[...] = ent_acc[...] / tokens
                mean = sw_acc[...] / tokens
                var = sw2_acc[...] / tokens - mean * mean
                std = jnp.sqrt(jnp.maximum(var, 0.0))
                cv_out[...] = jnp.mean(std / (mean + 1e-9)).reshape(1, 1)


def kernel(hidden_states, W1, b1, W2, b2):
    B, S, H = hidden_states.shape
    G = W2.shape[1]
    M = B * S
    x = hidden_states.reshape(M, H)

    mt = min(_M_T, M)
    nt = min(_N_T, H)
    kt = min(_K_T, H)
    nm, nn, nk = M // mt, H // nt, H // kt

    b1r = b1.reshape(1, H)
    b2r = b2.reshape(1, G)

    out_shape = (
        jax.ShapeDtypeStruct((M, G), jnp.float32),
        jax.ShapeDtypeStruct((M, G), jnp.float32),
        jax.ShapeDtypeStruct((1, 1), jnp.float32),
        jax.ShapeDtypeStruct((1, 1), jnp.float32),
    )
    grid = (nm, nn, nk)

    import functools
    body = functools.partial(_gate_kernel, nm=nm, nn=nn, nk=nk,
                             tokens=float(M), gates=G)

    weights, logits, ent, cv = pl.pallas_call(
        body,
        grid=grid,
        in_specs=[
            pl.BlockSpec((mt, kt), lambda m, n, k: (m, k)),
            pl.BlockSpec((kt, nt), lambda m, n, k: (k, n)),
            pl.BlockSpec((1, nt), lambda m, n, k: (0, n)),
            pl.BlockSpec((nt, G), lambda m, n, k: (n, 0)),
            pl.BlockSpec((1, G), lambda m, n, k: (0, 0)),
        ],
        out_specs=[
            pl.BlockSpec((mt, G), lambda m, n, k: (m, 0)),
            pl.BlockSpec((mt, G), lambda m, n, k: (m, 0)),
            pl.BlockSpec((1, 1), lambda m, n, k: (0, 0)),
            pl.BlockSpec((1, 1), lambda m, n, k: (0, 0)),
        ],
        out_shape=out_shape,
        scratch_shapes=[
            pltpu.VMEM((mt, nt), jnp.float32),
            pltpu.VMEM((mt, G), jnp.float32),
            pltpu.VMEM((1, 1), jnp.float32),
            pltpu.VMEM((1, G), jnp.float32),
            pltpu.VMEM((1, G), jnp.float32),
        ],
        compiler_params=pltpu.CompilerParams(
            dimension_semantics=("arbitrary", "arbitrary", "arbitrary"),
        ),
    )(x, W1, b1r, W2, b2r)

    return (weights.reshape(B, S, G), logits.reshape(B, S, G),
            ent.reshape(()), cv.reshape(()))


# speedup vs baseline: 1.5877x; 1.1922x over previous
import functools

import jax
import jax.numpy as jnp
from jax.experimental import pallas as pl
from jax.experimental.pallas import tpu as pltpu

_M_T = 256


def _gate_kernel(x_ref, w1_ref, b1_ref, w2_ref, b2_ref,
                 weights_out, logits_out, ent_out, cv_out,
                 ent_acc, sw_acc, sw2_acc,
                 *, nm, tokens, gates):
    m = pl.program_id(0)

    xb = x_ref[...].astype(jnp.bfloat16)
    h = jnp.dot(xb, w1_ref[...], preferred_element_type=jnp.float32)
    hb = (h + b1_ref[...]).astype(jnp.bfloat16)
    g = jax.nn.gelu(hb)
    logits = jnp.dot(g, w2_ref[...],
                     preferred_element_type=jnp.float32) + b2_ref[...]
    logits_out[...] = logits
    w = jax.nn.sigmoid(logits)
    weights_out[...] = w

    sw = jnp.sum(w, axis=0, keepdims=True)
    sw2 = jnp.sum(w * w, axis=0, keepdims=True)
    mx = jnp.max(logits, axis=-1, keepdims=True)
    e = jnp.exp(logits - mx)
    p = e / jnp.sum(e, axis=-1, keepdims=True)
    ent = -jnp.sum(p * jnp.log(p + 1e-9), axis=-1, keepdims=True)
    d = ent - jnp.log(jnp.float32(gates))
    e2 = jnp.sum(d * d).reshape(1, 1)

    @pl.when(m == 0)
    def _():
        sw_acc[...] = sw
        sw2_acc[...] = sw2
        ent_acc[...] = e2

    @pl.when(m != 0)
    def _():
        sw_acc[...] += sw
        sw2_acc[...] += sw2
        ent_acc[...] += e2

    @pl.when(m == nm - 1)
    def _():
        ent_out[...] = ent_acc[...] / tokens
        mean = sw_acc[...] / tokens
        var = sw2_acc[...] / tokens - mean * mean
        std = jnp.sqrt(jnp.maximum(var, 0.0))
        cv_out[...] = jnp.mean(std / (mean + 1e-9)).reshape(1, 1)


def kernel(hidden_states, W1, b1, W2, b2):
    B, S, H = hidden_states.shape
    G = W2.shape[1]
    M = B * S
    x = hidden_states.reshape(M, H)

    mt = min(_M_T, M)
    nm = M // mt

    w1b = W1.astype(jnp.bfloat16)
    w2b = W2.astype(jnp.bfloat16)
    b1r = b1.reshape(1, H)
    b2r = b2.reshape(1, G)

    out_shape = (
        jax.ShapeDtypeStruct((M, G), jnp.float32),
        jax.ShapeDtypeStruct((M, G), jnp.float32),
        jax.ShapeDtypeStruct((1, 1), jnp.float32),
        jax.ShapeDtypeStruct((1, 1), jnp.float32),
    )

    body = functools.partial(_gate_kernel, nm=nm, tokens=float(M), gates=G)

    weights, logits, ent, cv = pl.pallas_call(
        body,
        grid=(nm,),
        in_specs=[
            pl.BlockSpec((mt, H), lambda m: (m, 0)),
            pl.BlockSpec((H, H), lambda m: (0, 0)),
            pl.BlockSpec((1, H), lambda m: (0, 0)),
            pl.BlockSpec((H, G), lambda m: (0, 0)),
            pl.BlockSpec((1, G), lambda m: (0, 0)),
        ],
        out_specs=[
            pl.BlockSpec((mt, G), lambda m: (m, 0)),
            pl.BlockSpec((mt, G), lambda m: (m, 0)),
            pl.BlockSpec((1, 1), lambda m: (0, 0)),
            pl.BlockSpec((1, 1), lambda m: (0, 0)),
        ],
        out_shape=out_shape,
        scratch_shapes=[
            pltpu.VMEM((1, 1), jnp.float32),
            pltpu.VMEM((1, G), jnp.float32),
            pltpu.VMEM((1, G), jnp.float32),
        ],
        compiler_params=pltpu.CompilerParams(
            dimension_semantics=("arbitrary",),
        ),
    )(x, w1b, b1r, w2b, b2r)

    return (weights.reshape(B, S, G), logits.reshape(B, S, G),
            ent.reshape(()), cv.reshape(()))
